# baseline (speedup 1.0000x reference)
import jax
import jax.numpy as jnp
from jax.experimental import pallas as pl

_DISTANCE_THRESHOLD = 50.0
_NO_CODE_ID = -1
_BT = 512
_C = 128
_R = 8


def _nn_body(x_ref, codes_ref, out_ref):
    codes = codes_ref[...]
    k = codes.shape[0]
    cm2 = codes * (-2.0)
    csq = jnp.sum(codes * codes, axis=1)[None, :]
    tokens = x_ref.shape[0]
    for t in range(tokens // _BT):
        xt = x_ref[pl.ds(t * _BT, _BT), :]
        x_sq = jnp.sum(xt * xt, axis=1, keepdims=True)
        cross2 = jax.lax.dot_general(
            xt, cm2, (((1,), (1,)), ((), ())),
            preferred_element_type=jnp.float32,
        )
        for r in range(_BT // _R):
            xs = x_sq[r * _R:(r + 1) * _R, :]
            minv = (xs + csq[:, 0:_C]) + cross2[r * _R:(r + 1) * _R, 0:_C]
            mini = jnp.zeros((_R, _C), jnp.int32)
            for j in range(1, k // _C):
                d = (xs + csq[:, j * _C:(j + 1) * _C]) \
                    + cross2[r * _R:(r + 1) * _R, j * _C:(j + 1) * _C]
                better = d < minv
                minv = jnp.where(better, d, minv)
                mini = jnp.where(better, j, mini)
            lane = jax.lax.broadcasted_iota(jnp.int32, (_R, _C), 1)
            gmin = jnp.min(minv, axis=1)
            k_arr = mini * _C + lane
            k_cand = jnp.where(minv == gmin[:, None], k_arr, k)
            idx = jnp.min(k_cand, axis=1)
            out_ref[0, pl.ds(t * _BT + r * _R, _R)] = jnp.where(
                gmin <= _DISTANCE_THRESHOLD, idx, _NO_CODE_ID)


def kernel(x, codes):
    b, n, d = x.shape
    tokens = b * n
    xf = x.reshape(tokens, d)
    out = pl.pallas_call(
        _nn_body,
        out_shape=jax.ShapeDtypeStruct((1, tokens), jnp.int32),
    )(xf, codes)
    return out.reshape(b, n)


# speedup vs baseline: 1.0315x; 1.0315x over previous
import jax
import jax.numpy as jnp
from jax.experimental import pallas as pl

_DISTANCE_THRESHOLD = 50.0
_NO_CODE_ID = -1
_BT = 256
_C = 128


def _nn_body(x_ref, codes_ref, out_ref):
    codes = codes_ref[...]
    k = codes.shape[0]
    cm2 = codes * (-2.0)
    csq = jnp.sum(codes * codes, axis=1)[None, :]
    tokens = x_ref.shape[0]
    for t in range(tokens // _BT):
        xt = x_ref[pl.ds(t * _BT, _BT), :]
        x_sq = jnp.sum(xt * xt, axis=1, keepdims=True)
        cross2 = jax.lax.dot_general(
            xt, cm2, (((1,), (1,)), ((), ())),
            preferred_element_type=jnp.float32,
        )
        minv = (x_sq + csq[:, 0:_C]) + cross2[:, 0:_C]
        mini = jnp.zeros((_BT, _C), jnp.int32)
        for j in range(1, k // _C):
            d = (x_sq + csq[:, j * _C:(j + 1) * _C]) + cross2[:, j * _C:(j + 1) * _C]
            better = d < minv
            minv = jnp.where(better, d, minv)
            mini = jnp.where(better, j, mini)
        lane = jax.lax.broadcasted_iota(jnp.int32, (_BT, _C), 1)
        gmin = jnp.min(minv, axis=1)
        k_arr = mini * _C + lane
        k_cand = jnp.where(minv == gmin[:, None], k_arr, k)
        idx = jnp.min(k_cand, axis=1)
        out_ref[0, pl.ds(t * _BT, _BT)] = jnp.where(
            gmin <= _DISTANCE_THRESHOLD, idx, _NO_CODE_ID)


def kernel(x, codes):
    b, n, d = x.shape
    tokens = b * n
    xf = x.reshape(tokens, d)
    out = pl.pallas_call(
        _nn_body,
        out_shape=jax.ShapeDtypeStruct((1, tokens), jnp.int32),
    )(xf, codes)
    return out.reshape(b, n)
